# Initial kernel scaffold; baseline (speedup 1.0000x reference)
import jax
import jax.numpy as jnp
from jax.experimental import pallas as pl


def kernel(x, edge_index, W1, b1, W2, b2):
    raise NotImplementedError("write your pallas kernel here")


# speedup vs baseline: 17.9309x; 17.9309x over previous
import functools

import jax
import jax.numpy as jnp
from jax import lax
from jax.experimental import pallas as pl
from jax.experimental.pallas import tpu as pltpu
from jax.experimental.pallas import tpu_sc as plsc

N = 10000
E = 320000
D = 128
C = 40
CP = 48

NC = 2
NS = 16
NW = NC * NS
K = 80
EPW = E // NW
NCHUNK = EPW // K
STRIPE = N // NS

_MESH = plsc.VectorSubcoreMesh(
    core_axis_name="c", subcore_axis_name="s", num_cores=NC, num_subcores=NS)
_SC_PARAMS = pltpu.CompilerParams(use_tc_tiling_on_sc=False)


def _deg_body(dst_hbm, ones_hbm, z_hbm, degp_hbm, dst_v, ones_v, acc):
  cid = lax.axis_index("c")
  sid = lax.axis_index("s")
  wid = sid * NC + cid
  pltpu.sync_copy(z_hbm, acc.at[pl.ds(sid * STRIPE, STRIPE)])
  pltpu.sync_copy(ones_hbm, ones_v)
  pltpu.sync_copy(dst_hbm.at[pl.ds(wid * NCHUNK, NCHUNK)], dst_v)
  plsc.subcore_barrier()

  def body(j, carry):
    pltpu.sync_copy(ones_v, acc.at[dst_v.at[j]], add=True)
    return carry

  lax.fori_loop(0, NCHUNK, body, 0)
  plsc.subcore_barrier()
  pltpu.sync_copy(acc.at[pl.ds(sid * STRIPE, STRIPE)],
                  degp_hbm.at[cid, pl.ds(sid * STRIPE, STRIPE)])


def _deg_call(dst_resh, ones8, z8):
  return pl.kernel(
      _deg_body,
      out_type=jax.ShapeDtypeStruct((NC, N, 8), jnp.float32),
      mesh=_MESH,
      scratch_types=[
          pltpu.VMEM((NCHUNK, K), jnp.int32),
          pltpu.VMEM((K, 8), jnp.float32),
          pltpu.VMEM_SHARED((N, 8), jnp.float32),
      ],
      compiler_params=_SC_PARAMS,
  )(dst_resh, ones8, z8)


def _prop_body(g_hbm, src_hbm, dst_hbm, z_hbm, out_hbm,
               src_v, dst_v, rows_v, acc, gsem):
  cid = lax.axis_index("c")
  sid = lax.axis_index("s")
  wid = sid * NC + cid
  pltpu.sync_copy(z_hbm, acc.at[pl.ds(sid * STRIPE, STRIPE)])
  pltpu.sync_copy(src_hbm.at[pl.ds(wid * NCHUNK, NCHUNK)], src_v)
  pltpu.sync_copy(dst_hbm.at[pl.ds(wid * NCHUNK, NCHUNK)], dst_v)
  plsc.subcore_barrier()

  def body(j, carry):
    pltpu.async_copy(g_hbm.at[src_v.at[j]], rows_v, gsem).wait()
    pltpu.sync_copy(rows_v, acc.at[dst_v.at[j]], add=True)
    return carry

  lax.fori_loop(0, NCHUNK, body, 0)
  plsc.subcore_barrier()
  pltpu.sync_copy(acc.at[pl.ds(sid * STRIPE, STRIPE)],
                  out_hbm.at[cid, pl.ds(sid * STRIPE, STRIPE)])


def _prop_call(g, src_resh, dst_resh, zd, d):
  return pl.kernel(
      _prop_body,
      out_type=jax.ShapeDtypeStruct((NC, N, d), jnp.float32),
      mesh=_MESH,
      scratch_types=[
          pltpu.VMEM((NCHUNK, K), jnp.int32),
          pltpu.VMEM((NCHUNK, K), jnp.int32),
          pltpu.VMEM((K, d), jnp.float32),
          pltpu.VMEM_SHARED((N, d), jnp.float32),
          pltpu.SemaphoreType.DMA,
      ],
      compiler_params=_SC_PARAMS,
  )(g, src_resh, dst_resh, zd)


R = 1000


def _dinv_block(degp):
  deg = degp[0, :, 0] + degp[1, :, 0]
  return lax.rsqrt(jnp.maximum(deg, 1.0))


def _tc1_kernel(x_ref, w1_ref, b1_ref, degp_ref, g1_ref):
  dinv = _dinv_block(degp_ref[...])
  h = jnp.dot(x_ref[...], w1_ref[...], preferred_element_type=jnp.float32)
  g1_ref[...] = (h + b1_ref[...]) * dinv[:, None]


def _tc2_kernel(s1p_ref, degp_ref, w2_ref, b2_ref, g2_ref):
  dinv = _dinv_block(degp_ref[...])
  s1 = s1p_ref[0] + s1p_ref[1]
  h = jnp.maximum(s1 * dinv[:, None], 0.0)
  out = jnp.dot(h, w2_ref[...], preferred_element_type=jnp.float32)
  g2_ref[...] = (out + b2_ref[...]) * dinv[:, None]


def _tc3_kernel(s2p_ref, degp_ref, out_ref):
  dinv = _dinv_block(degp_ref[...])
  z = (s2p_ref[0] + s2p_ref[1]) * dinv[:, None]
  z = z[:, :C]
  m = jnp.max(z, axis=1, keepdims=True)
  e = jnp.exp(z - m)
  lse = jnp.log(jnp.sum(e, axis=1, keepdims=True))
  out_ref[...] = z - m - lse


def _tc1(x, W1, b1r, degp):
  grid = (N // R,)
  return pl.pallas_call(
      _tc1_kernel,
      grid=grid,
      in_specs=[
          pl.BlockSpec((R, D), lambda i: (i, 0)),
          pl.BlockSpec((D, D), lambda i: (0, 0)),
          pl.BlockSpec((1, D), lambda i: (0, 0)),
          pl.BlockSpec((NC, R, 8), lambda i: (0, i, 0)),
      ],
      out_specs=pl.BlockSpec((R, D), lambda i: (i, 0)),
      out_shape=jax.ShapeDtypeStruct((N, D), jnp.float32),
  )(x, W1, b1r, degp)


def _tc2(s1p, degp, W2p, b2p):
  grid = (N // R,)
  return pl.pallas_call(
      _tc2_kernel,
      grid=grid,
      in_specs=[
          pl.BlockSpec((NC, R, D), lambda i: (0, i, 0)),
          pl.BlockSpec((NC, R, 8), lambda i: (0, i, 0)),
          pl.BlockSpec((D, CP), lambda i: (0, 0)),
          pl.BlockSpec((1, CP), lambda i: (0, 0)),
      ],
      out_specs=pl.BlockSpec((R, CP), lambda i: (i, 0)),
      out_shape=jax.ShapeDtypeStruct((N, CP), jnp.float32),
  )(s1p, degp, W2p, b2p)


def _tc3(s2p, degp):
  grid = (N // R,)
  return pl.pallas_call(
      _tc3_kernel,
      grid=grid,
      in_specs=[
          pl.BlockSpec((NC, R, CP), lambda i: (0, i, 0)),
          pl.BlockSpec((NC, R, 8), lambda i: (0, i, 0)),
      ],
      out_specs=pl.BlockSpec((R, C), lambda i: (i, 0)),
      out_shape=jax.ShapeDtypeStruct((N, C), jnp.float32),
  )(s2p, degp)


def kernel(x, edge_index, W1, b1, W2, b2):
  src_resh = edge_index[0].reshape(E // K, K)
  dst_resh = edge_index[1].reshape(E // K, K)
  b1r = b1.reshape(1, D)
  W2p = jnp.pad(W2, ((0, 0), (0, CP - C)))
  b2p = jnp.pad(b2, (0, CP - C)).reshape(1, CP)
  ones8 = jnp.ones((K, 8), jnp.float32)
  z8 = jnp.zeros((STRIPE, 8), jnp.float32)
  z128 = jnp.zeros((STRIPE, D), jnp.float32)
  z48 = jnp.zeros((STRIPE, CP), jnp.float32)

  degp = _deg_call(dst_resh, ones8, z8)
  g1 = _tc1(x, W1, b1r, degp)
  s1p = _prop_call(g1, src_resh, dst_resh, z128, D)
  g2 = _tc2(s1p, degp, W2p, b2p)
  s2p = _prop_call(g2, src_resh, dst_resh, z48, CP)
  return _tc3(s2p, degp)


# speedup vs baseline: 26.4075x; 1.4727x over previous
import functools

import jax
import jax.numpy as jnp
from jax import lax
from jax.experimental import pallas as pl
from jax.experimental.pallas import tpu as pltpu
from jax.experimental.pallas import tpu_sc as plsc

N = 10000
E = 320000
D = 128
C = 40
CP = 48

NC = 2
NS = 16
NW = NC * NS
K = 80
EPW = E // NW
NCHUNK = EPW // K
STRIPE = N // NS

_MESH = plsc.VectorSubcoreMesh(
    core_axis_name="c", subcore_axis_name="s", num_cores=NC, num_subcores=NS)
_SC_PARAMS = pltpu.CompilerParams(use_tc_tiling_on_sc=False)


def _deg_body(dst_hbm, ones_hbm, z_hbm, degp_hbm, dst_v, ones_v, acc):
  cid = lax.axis_index("c")
  sid = lax.axis_index("s")
  wid = sid * NC + cid
  pltpu.sync_copy(z_hbm, acc.at[pl.ds(sid * STRIPE, STRIPE)])
  pltpu.sync_copy(ones_hbm, ones_v)
  pltpu.sync_copy(dst_hbm.at[pl.ds(wid * NCHUNK, NCHUNK)], dst_v)
  plsc.subcore_barrier()

  def body(j, carry):
    pltpu.sync_copy(ones_v, acc.at[dst_v.at[j]], add=True)
    return carry

  lax.fori_loop(0, NCHUNK, body, 0)
  plsc.subcore_barrier()
  pltpu.sync_copy(acc.at[pl.ds(sid * STRIPE, STRIPE)],
                  degp_hbm.at[cid, pl.ds(sid * STRIPE, STRIPE)])


def _deg_call(dst_resh, ones8, z8):
  return pl.kernel(
      _deg_body,
      out_type=jax.ShapeDtypeStruct((NC, N, 8), jnp.float32),
      mesh=_MESH,
      scratch_types=[
          pltpu.VMEM((NCHUNK, K), jnp.int32),
          pltpu.VMEM((K, 8), jnp.float32),
          pltpu.VMEM_SHARED((N, 8), jnp.float32),
      ],
      compiler_params=_SC_PARAMS,
  )(dst_resh, ones8, z8)


def _prop_body(g_hbm, src_hbm, dst_hbm, z_hbm, out_hbm,
               src_v, dst_v, rows0, rows1, acc, gs0, gs1):
  cid = lax.axis_index("c")
  sid = lax.axis_index("s")
  wid = sid * NC + cid
  pltpu.sync_copy(z_hbm, acc.at[pl.ds(sid * STRIPE, STRIPE)])
  pltpu.sync_copy(src_hbm.at[pl.ds(wid * NCHUNK, NCHUNK)], src_v)
  pltpu.sync_copy(dst_hbm.at[pl.ds(wid * NCHUNK, NCHUNK)], dst_v)
  plsc.subcore_barrier()

  pltpu.async_copy(g_hbm.at[src_v.at[0]], rows0, gs0)
  pltpu.async_copy(g_hbm.at[src_v.at[1]], rows1, gs1)

  def chunk(j, buf, gs):
    pltpu.make_async_copy(g_hbm.at[src_v.at[j]], buf, gs).wait()
    pltpu.sync_copy(buf, acc.at[dst_v.at[j]], add=True)

    @pl.when(j + 2 < NCHUNK)
    def _():
      pltpu.async_copy(g_hbm.at[src_v.at[j + 2]], buf, gs)

  def body(j, carry):
    @pl.when(lax.rem(j, 2) == 0)
    def _():
      chunk(j, rows0, gs0)

    @pl.when(lax.rem(j, 2) == 1)
    def _():
      chunk(j, rows1, gs1)

    return carry

  lax.fori_loop(0, NCHUNK, body, 0)
  plsc.subcore_barrier()
  pltpu.sync_copy(acc.at[pl.ds(sid * STRIPE, STRIPE)],
                  out_hbm.at[cid, pl.ds(sid * STRIPE, STRIPE)])


def _prop_call(g, src_resh, dst_resh, zd, d):
  return pl.kernel(
      _prop_body,
      out_type=jax.ShapeDtypeStruct((NC, N, d), jnp.float32),
      mesh=_MESH,
      scratch_types=[
          pltpu.VMEM((NCHUNK, K), jnp.int32),
          pltpu.VMEM((NCHUNK, K), jnp.int32),
          pltpu.VMEM((K, d), jnp.float32),
          pltpu.VMEM((K, d), jnp.float32),
          pltpu.VMEM_SHARED((N, d), jnp.float32),
          pltpu.SemaphoreType.DMA,
          pltpu.SemaphoreType.DMA,
      ],
      compiler_params=_SC_PARAMS,
  )(g, src_resh, dst_resh, zd)


R = 1000


def _dinv_block(degp):
  deg = degp[0, :, 0] + degp[1, :, 0]
  return lax.rsqrt(jnp.maximum(deg, 1.0))


def _tc1_kernel(x_ref, w1_ref, b1_ref, degp_ref, g1_ref):
  dinv = _dinv_block(degp_ref[...])
  h = jnp.dot(x_ref[...], w1_ref[...], preferred_element_type=jnp.float32)
  g1_ref[...] = (h + b1_ref[...]) * dinv[:, None]


def _tc2_kernel(s1p_ref, degp_ref, w2_ref, b2_ref, g2_ref):
  dinv = _dinv_block(degp_ref[...])
  s1 = s1p_ref[0] + s1p_ref[1]
  h = jnp.maximum(s1 * dinv[:, None], 0.0)
  out = jnp.dot(h, w2_ref[...], preferred_element_type=jnp.float32)
  g2_ref[...] = (out + b2_ref[...]) * dinv[:, None]


def _tc3_kernel(s2p_ref, degp_ref, out_ref):
  dinv = _dinv_block(degp_ref[...])
  z = (s2p_ref[0] + s2p_ref[1]) * dinv[:, None]
  z = z[:, :C]
  m = jnp.max(z, axis=1, keepdims=True)
  e = jnp.exp(z - m)
  lse = jnp.log(jnp.sum(e, axis=1, keepdims=True))
  out_ref[...] = z - m - lse


def _tc1(x, W1, b1r, degp):
  grid = (N // R,)
  return pl.pallas_call(
      _tc1_kernel,
      grid=grid,
      in_specs=[
          pl.BlockSpec((R, D), lambda i: (i, 0)),
          pl.BlockSpec((D, D), lambda i: (0, 0)),
          pl.BlockSpec((1, D), lambda i: (0, 0)),
          pl.BlockSpec((NC, R, 8), lambda i: (0, i, 0)),
      ],
      out_specs=pl.BlockSpec((R, D), lambda i: (i, 0)),
      out_shape=jax.ShapeDtypeStruct((N, D), jnp.float32),
  )(x, W1, b1r, degp)


def _tc2(s1p, degp, W2p, b2p):
  grid = (N // R,)
  return pl.pallas_call(
      _tc2_kernel,
      grid=grid,
      in_specs=[
          pl.BlockSpec((NC, R, D), lambda i: (0, i, 0)),
          pl.BlockSpec((NC, R, 8), lambda i: (0, i, 0)),
          pl.BlockSpec((D, CP), lambda i: (0, 0)),
          pl.BlockSpec((1, CP), lambda i: (0, 0)),
      ],
      out_specs=pl.BlockSpec((R, CP), lambda i: (i, 0)),
      out_shape=jax.ShapeDtypeStruct((N, CP), jnp.float32),
  )(s1p, degp, W2p, b2p)


def _tc3(s2p, degp):
  grid = (N // R,)
  return pl.pallas_call(
      _tc3_kernel,
      grid=grid,
      in_specs=[
          pl.BlockSpec((NC, R, CP), lambda i: (0, i, 0)),
          pl.BlockSpec((NC, R, 8), lambda i: (0, i, 0)),
      ],
      out_specs=pl.BlockSpec((R, C), lambda i: (i, 0)),
      out_shape=jax.ShapeDtypeStruct((N, C), jnp.float32),
  )(s2p, degp)


def kernel(x, edge_index, W1, b1, W2, b2):
  src_resh = edge_index[0].reshape(E // K, K)
  dst_resh = edge_index[1].reshape(E // K, K)
  b1r = b1.reshape(1, D)
  W2p = jnp.pad(W2, ((0, 0), (0, CP - C)))
  b2p = jnp.pad(b2, (0, CP - C)).reshape(1, CP)
  ones8 = jnp.ones((K, 8), jnp.float32)
  z8 = jnp.zeros((STRIPE, 8), jnp.float32)
  z128 = jnp.zeros((STRIPE, D), jnp.float32)
  z48 = jnp.zeros((STRIPE, CP), jnp.float32)

  degp = _deg_call(dst_resh, ones8, z8)
  g1 = _tc1(x, W1, b1r, degp)
  s1p = _prop_call(g1, src_resh, dst_resh, z128, D)
  g2 = _tc2(s1p, degp, W2p, b2p)
  s2p = _prop_call(g2, src_resh, dst_resh, z48, CP)
  return _tc3(s2p, degp)


# speedup vs baseline: 28.4028x; 1.0756x over previous
import functools

import jax
import jax.numpy as jnp
from jax import lax
from jax.experimental import pallas as pl
from jax.experimental.pallas import tpu as pltpu
from jax.experimental.pallas import tpu_sc as plsc

N = 10000
E = 320000
D = 128
C = 40
CP = 48

NC = 2
NS = 16
NW = NC * NS
K = 80
EPW = E // NW
NCHUNK = EPW // K
STRIPE = N // NS

_MESH = plsc.VectorSubcoreMesh(
    core_axis_name="c", subcore_axis_name="s", num_cores=NC, num_subcores=NS)
_SC_PARAMS = pltpu.CompilerParams(use_tc_tiling_on_sc=False)


def _deg_body(edges_hbm, ones_hbm, z_hbm, degp_hbm, dst_v, ones_v, acc, ssem):
  cid = lax.axis_index("c")
  sid = lax.axis_index("s")
  wid = sid * NC + cid
  pltpu.sync_copy(z_hbm, acc.at[pl.ds(sid * STRIPE, STRIPE)])
  pltpu.sync_copy(ones_hbm, ones_v)
  pltpu.sync_copy(edges_hbm.at[1, pl.ds(wid * NCHUNK, NCHUNK)], dst_v)
  plsc.subcore_barrier()

  def body(j, carry):
    pltpu.async_copy(ones_v, acc.at[dst_v.at[j]], ssem, add=True)
    return carry

  lax.fori_loop(0, NCHUNK, body, 0)

  def drain(j, carry):
    pltpu.make_async_copy(ones_v, acc.at[dst_v.at[j]], ssem).wait()
    return carry

  lax.fori_loop(0, NCHUNK, drain, 0)
  plsc.subcore_barrier()
  pltpu.sync_copy(acc.at[pl.ds(sid * STRIPE, STRIPE)],
                  degp_hbm.at[cid, pl.ds(sid * STRIPE, STRIPE)])


def _deg_call(edges3, ones8, z8):
  return pl.kernel(
      _deg_body,
      out_type=jax.ShapeDtypeStruct((NC, N, 8), jnp.float32),
      mesh=_MESH,
      scratch_types=[
          pltpu.VMEM((NCHUNK, K), jnp.int32),
          pltpu.VMEM((K, 8), jnp.float32),
          pltpu.VMEM_SHARED((N, 8), jnp.float32),
          pltpu.SemaphoreType.DMA,
      ],
      compiler_params=_SC_PARAMS,
  )(edges3, ones8, z8)


def _prop_body(g_hbm, edges_hbm, z_hbm, out_hbm,
               src_v, dst_v, rows0, rows1, acc, gs0, gs1):
  cid = lax.axis_index("c")
  sid = lax.axis_index("s")
  wid = sid * NC + cid
  pltpu.sync_copy(z_hbm, acc.at[pl.ds(sid * STRIPE, STRIPE)])
  pltpu.sync_copy(edges_hbm.at[0, pl.ds(wid * NCHUNK, NCHUNK)], src_v)
  pltpu.sync_copy(edges_hbm.at[1, pl.ds(wid * NCHUNK, NCHUNK)], dst_v)
  plsc.subcore_barrier()

  pltpu.async_copy(g_hbm.at[src_v.at[0]], rows0, gs0)
  pltpu.async_copy(g_hbm.at[src_v.at[1]], rows1, gs1)

  def chunk(j, buf, gs):
    pltpu.make_async_copy(g_hbm.at[src_v.at[j]], buf, gs).wait()
    pltpu.sync_copy(buf, acc.at[dst_v.at[j]], add=True)

    @pl.when(j + 2 < NCHUNK)
    def _():
      pltpu.async_copy(g_hbm.at[src_v.at[j + 2]], buf, gs)

  def body(j, carry):
    @pl.when(lax.rem(j, 2) == 0)
    def _():
      chunk(j, rows0, gs0)

    @pl.when(lax.rem(j, 2) == 1)
    def _():
      chunk(j, rows1, gs1)

    return carry

  lax.fori_loop(0, NCHUNK, body, 0)
  plsc.subcore_barrier()
  pltpu.sync_copy(acc.at[pl.ds(sid * STRIPE, STRIPE)],
                  out_hbm.at[cid, pl.ds(sid * STRIPE, STRIPE)])


def _prop_call(g, edges3, zd, d):
  return pl.kernel(
      _prop_body,
      out_type=jax.ShapeDtypeStruct((NC, N, d), jnp.float32),
      mesh=_MESH,
      scratch_types=[
          pltpu.VMEM((NCHUNK, K), jnp.int32),
          pltpu.VMEM((NCHUNK, K), jnp.int32),
          pltpu.VMEM((K, d), jnp.float32),
          pltpu.VMEM((K, d), jnp.float32),
          pltpu.VMEM_SHARED((N, d), jnp.float32),
          pltpu.SemaphoreType.DMA,
          pltpu.SemaphoreType.DMA,
      ],
      compiler_params=_SC_PARAMS,
  )(g, edges3, zd)


R = 1000


def _dinv_block(degp):
  deg = degp[0, :, 0] + degp[1, :, 0]
  return lax.rsqrt(jnp.maximum(deg, 1.0))


def _tc1_kernel(x_ref, w1_ref, b1_ref, degp_ref, g1_ref):
  dinv = _dinv_block(degp_ref[...])
  h = jnp.dot(x_ref[...], w1_ref[...], preferred_element_type=jnp.float32)
  g1_ref[...] = (h + b1_ref[...]) * dinv[:, None]


def _tc2_kernel(s1p_ref, degp_ref, w2_ref, b2_ref, g2_ref):
  dinv = _dinv_block(degp_ref[...])
  s1 = s1p_ref[0] + s1p_ref[1]
  h = jnp.maximum(s1 * dinv[:, None], 0.0)
  out = jnp.dot(h, w2_ref[...], preferred_element_type=jnp.float32)
  g2_ref[...] = (out + b2_ref[...]) * dinv[:, None]


def _tc3_kernel(s2p_ref, degp_ref, out_ref):
  dinv = _dinv_block(degp_ref[...])
  z = (s2p_ref[0] + s2p_ref[1]) * dinv[:, None]
  z = z[:, :C]
  m = jnp.max(z, axis=1, keepdims=True)
  e = jnp.exp(z - m)
  lse = jnp.log(jnp.sum(e, axis=1, keepdims=True))
  out_ref[...] = z - m - lse


def _tc1(x, W1, b1r, degp):
  return pl.pallas_call(
      _tc1_kernel,
      out_shape=jax.ShapeDtypeStruct((N, D), jnp.float32),
  )(x, W1, b1r, degp)


def _tc2(s1p, degp, W2p, b2p):
  return pl.pallas_call(
      _tc2_kernel,
      out_shape=jax.ShapeDtypeStruct((N, CP), jnp.float32),
  )(s1p, degp, W2p, b2p)


def _tc3(s2p, degp):
  return pl.pallas_call(
      _tc3_kernel,
      out_shape=jax.ShapeDtypeStruct((N, C), jnp.float32),
  )(s2p, degp)


def kernel(x, edge_index, W1, b1, W2, b2):
  edges3 = edge_index.reshape(2, E // K, K)
  b1r = b1.reshape(1, D)
  W2p = jnp.pad(W2, ((0, 0), (0, CP - C)))
  b2p = jnp.pad(b2, (0, CP - C)).reshape(1, CP)
  ones8 = jnp.ones((K, 8), jnp.float32)
  z8 = jnp.zeros((STRIPE, 8), jnp.float32)
  z128 = jnp.zeros((STRIPE, D), jnp.float32)
  z48 = jnp.zeros((STRIPE, CP), jnp.float32)

  degp = _deg_call(edges3, ones8, z8)
  g1 = _tc1(x, W1, b1r, degp)
  s1p = _prop_call(g1, edges3, z128, D)
  g2 = _tc2(s1p, degp, W2p, b2p)
  s2p = _prop_call(g2, edges3, z48, CP)
  return _tc3(s2p, degp)


# speedup vs baseline: 30.4509x; 1.0721x over previous
import jax
import jax.numpy as jnp
import numpy as np
from jax import lax
from jax.experimental import pallas as pl
from jax.experimental.pallas import tpu as pltpu
from jax.experimental.pallas import tpu_sc as plsc

N = 10000
E = 320000
D = 128
C = 40
CP = 48

NC = 2
NS = 16
NW = NC * NS
K = 128
EP = 327680
NCHUNK = EP // (NW * K)
NP = 10016
SP = NP // NS

_MESH = plsc.VectorSubcoreMesh(
    core_axis_name="c", subcore_axis_name="s", num_cores=NC, num_subcores=NS)
_SC_PARAMS = pltpu.CompilerParams(use_tc_tiling_on_sc=False)


def _deg_body(edges_hbm, ones_hbm, z_hbm, degp_hbm, dst_v, ones_v, acc, ssem):
  cid = lax.axis_index("c")
  sid = lax.axis_index("s")
  wid = sid * NC + cid
  pltpu.sync_copy(z_hbm, acc.at[pl.ds(sid * SP, SP)])
  pltpu.sync_copy(ones_hbm, ones_v)
  pltpu.sync_copy(edges_hbm.at[1, pl.ds(wid * NCHUNK, NCHUNK)], dst_v)
  plsc.subcore_barrier()

  def body(j, carry):
    pltpu.async_copy(ones_v, acc.at[dst_v.at[j]], ssem, add=True)
    return carry

  lax.fori_loop(0, NCHUNK, body, 0)

  def drain(j, carry):
    pltpu.make_async_copy(ones_v, acc.at[dst_v.at[j]], ssem).wait()
    return carry

  lax.fori_loop(0, NCHUNK, drain, 0)
  plsc.subcore_barrier()
  pltpu.sync_copy(acc.at[pl.ds(sid * SP, SP)],
                  degp_hbm.at[cid, pl.ds(sid * SP, SP)])


def _deg_call(edges3, ones8, z8):
  return pl.kernel(
      _deg_body,
      out_type=jax.ShapeDtypeStruct((NC, NP, 8), jnp.float32),
      mesh=_MESH,
      scratch_types=[
          pltpu.VMEM((NCHUNK, K), jnp.int32),
          pltpu.VMEM((K, 8), jnp.float32),
          pltpu.VMEM_SHARED((NP, 8), jnp.float32),
          pltpu.SemaphoreType.DMA,
      ],
      compiler_params=_SC_PARAMS,
  )(edges3, ones8, z8)


def _make_prop_body(nstage):
  hc = NCHUNK // nstage

  def _prop_body(g_hbm, edges_hbm, z_hbm, out_hbm,
                 src_v, dst_v, rows0, rows1, acc, gs0, gs1):
    cid = lax.axis_index("c")
    sid = lax.axis_index("s")
    wid = sid * NC + cid
    pltpu.sync_copy(z_hbm, acc.at[pl.ds(sid * SP, SP)])

    for h in range(nstage):
      base = wid * NCHUNK + h * hc
      pltpu.sync_copy(edges_hbm.at[0, pl.ds(base, hc)], src_v)
      pltpu.sync_copy(edges_hbm.at[1, pl.ds(base, hc)], dst_v)
      if h == 0:
        plsc.subcore_barrier()
      pltpu.async_copy(g_hbm.at[src_v.at[0]], rows0, gs0)
      pltpu.async_copy(g_hbm.at[src_v.at[1]], rows1, gs1)

      def chunk(j, buf, gs):
        pltpu.make_async_copy(g_hbm.at[src_v.at[j]], buf, gs).wait()
        pltpu.sync_copy(buf, acc.at[dst_v.at[j]], add=True)

        @pl.when(j + 2 < hc)
        def _():
          pltpu.async_copy(g_hbm.at[src_v.at[j + 2]], buf, gs)

      def body(j, carry):
        @pl.when(lax.rem(j, 2) == 0)
        def _():
          chunk(j, rows0, gs0)

        @pl.when(lax.rem(j, 2) == 1)
        def _():
          chunk(j, rows1, gs1)

        return carry

      lax.fori_loop(0, hc, body, 0)

    plsc.subcore_barrier()
    pltpu.sync_copy(acc.at[pl.ds(sid * SP, SP)],
                    out_hbm.at[cid, pl.ds(sid * SP, SP)])

  return _prop_body


def _prop_call(g, edges3, zd, d):
  nstage = 2 if d == D else 1
  hc = NCHUNK // nstage
  return pl.kernel(
      _make_prop_body(nstage),
      out_type=jax.ShapeDtypeStruct((NC, NP, d), jnp.float32),
      mesh=_MESH,
      scratch_types=[
          pltpu.VMEM((hc, K), jnp.int32),
          pltpu.VMEM((hc, K), jnp.int32),
          pltpu.VMEM((K, d), jnp.float32),
          pltpu.VMEM((K, d), jnp.float32),
          pltpu.VMEM_SHARED((NP, d), jnp.float32),
          pltpu.SemaphoreType.DMA,
          pltpu.SemaphoreType.DMA,
      ],
      compiler_params=_SC_PARAMS,
  )(g, edges3, zd)


def _dinv_n(degp):
  deg = degp[0, :N, 0] + degp[1, :N, 0]
  return lax.rsqrt(jnp.maximum(deg, 1.0))


def _tc0_kernel(x_ref, w1_ref, b1_ref, h1_ref):
  h = jnp.dot(x_ref[...], w1_ref[...], preferred_element_type=jnp.float32)
  h1_ref[...] = h + b1_ref[...]


def _tc1_kernel(h1_ref, degp_ref, g1_ref):
  g1_ref[...] = h1_ref[...] * _dinv_n(degp_ref[...])[:, None]


def _tc2_kernel(s1p_ref, degp_ref, w2_ref, b2_ref, g2_ref):
  dinv = _dinv_n(degp_ref[...])
  s1 = s1p_ref[0, :N] + s1p_ref[1, :N]
  h = jnp.maximum(s1 * dinv[:, None], 0.0)
  out = jnp.dot(h, w2_ref[...], preferred_element_type=jnp.float32)
  g2_ref[...] = (out + b2_ref[...]) * dinv[:, None]


def _tc3_kernel(s2p_ref, degp_ref, out_ref):
  dinv = _dinv_n(degp_ref[...])
  z = (s2p_ref[0, :N] + s2p_ref[1, :N]) * dinv[:, None]
  z = z[:, :C]
  m = jnp.max(z, axis=1, keepdims=True)
  e = jnp.exp(z - m)
  lse = jnp.log(jnp.sum(e, axis=1, keepdims=True))
  out_ref[...] = z - m - lse


def _tc0(x, W1, b1r):
  return pl.pallas_call(
      _tc0_kernel,
      out_shape=jax.ShapeDtypeStruct((N, D), jnp.float32),
  )(x, W1, b1r)


def _tc1(h1, degp):
  return pl.pallas_call(
      _tc1_kernel,
      out_shape=jax.ShapeDtypeStruct((N, D), jnp.float32),
  )(h1, degp)


def _tc2(s1p, degp, W2p, b2p):
  return pl.pallas_call(
      _tc2_kernel,
      out_shape=jax.ShapeDtypeStruct((N, CP), jnp.float32),
  )(s1p, degp, W2p, b2p)


def _tc3(s2p, degp):
  return pl.pallas_call(
      _tc3_kernel,
      out_shape=jax.ShapeDtypeStruct((N, C), jnp.float32),
  )(s2p, degp)


_PAD_EDGES = np.stack([(np.arange(EP - E) * 131) % N,
                       N + (np.arange(EP - E) % 16)]).astype(np.int32)


def kernel(x, edge_index, W1, b1, W2, b2):
  pad = jnp.asarray(_PAD_EDGES)
  edges3 = jnp.concatenate([edge_index, pad], axis=1).reshape(2, EP // K, K)
  b1r = b1.reshape(1, D)
  W2p = jnp.pad(W2, ((0, 0), (0, CP - C)))
  b2p = jnp.pad(b2, (0, CP - C)).reshape(1, CP)
  ones8 = jnp.ones((K, 8), jnp.float32)
  z8 = jnp.zeros((SP, 8), jnp.float32)
  z128 = jnp.zeros((SP, D), jnp.float32)
  z48 = jnp.zeros((SP, CP), jnp.float32)

  h1 = _tc0(x, W1, b1r)
  degp = _deg_call(edges3, ones8, z8)
  g1 = _tc1(h1, degp)
  s1p = _prop_call(g1, edges3, z128, D)
  g2 = _tc2(s1p, degp, W2p, b2p)
  s2p = _prop_call(g2, edges3, z48, CP)
  return _tc3(s2p, degp)


# speedup vs baseline: 33.3728x; 1.0960x over previous
import jax
import jax.numpy as jnp
import numpy as np
from jax import lax
from jax.experimental import pallas as pl
from jax.experimental.pallas import tpu as pltpu
from jax.experimental.pallas import tpu_sc as plsc

N = 10000
E = 320000
D = 128
C = 40
CP = 48

NC = 2
NS = 16
NW = NC * NS
K = 128
EP = 327680
NCHUNK = EP // (NW * K)
NP = 10016
SP = NP // NS
R = 2000

_MESH = plsc.VectorSubcoreMesh(
    core_axis_name="c", subcore_axis_name="s", num_cores=NC, num_subcores=NS)
_SC_PARAMS = pltpu.CompilerParams(use_tc_tiling_on_sc=False)


def _deg_body(edges_hbm, ones_hbm, z_hbm, degp_hbm, dst_v, ones_v, acc, ssem):
  cid = lax.axis_index("c")
  sid = lax.axis_index("s")
  wid = sid * NC + cid
  pltpu.sync_copy(z_hbm, acc.at[pl.ds(sid * SP, SP)])
  pltpu.sync_copy(ones_hbm, ones_v)
  pltpu.sync_copy(edges_hbm.at[1, pl.ds(wid * NCHUNK, NCHUNK)], dst_v)
  plsc.subcore_barrier()

  def body(j, carry):
    pltpu.async_copy(ones_v, acc.at[dst_v.at[j]], ssem, add=True)
    return carry

  lax.fori_loop(0, NCHUNK, body, 0)

  def drain(j, carry):
    pltpu.make_async_copy(ones_v, acc.at[dst_v.at[j]], ssem).wait()
    return carry

  lax.fori_loop(0, NCHUNK, drain, 0)
  plsc.subcore_barrier()
  pltpu.sync_copy(acc.at[pl.ds(sid * SP, SP)],
                  degp_hbm.at[cid, pl.ds(sid * SP, SP), pl.ds(0, 8)])


def _deg_call(edges3, ones8, z8):
  return pl.kernel(
      _deg_body,
      out_type=jax.ShapeDtypeStruct((NC, NP, D), jnp.float32),
      mesh=_MESH,
      scratch_types=[
          pltpu.VMEM((NCHUNK, K), jnp.int32),
          pltpu.VMEM((K, 8), jnp.float32),
          pltpu.VMEM_SHARED((NP, 8), jnp.float32),
          pltpu.SemaphoreType.DMA,
      ],
      compiler_params=_SC_PARAMS,
  )(edges3, ones8, z8)


def _prop128_body(g_hbm, edges_hbm, z_hbm, out_hbm,
                  src_v, dst_v, rows0, rows1, acc, gs0, gs1):
  cid = lax.axis_index("c")
  sid = lax.axis_index("s")
  wid = sid * NC + cid
  pltpu.sync_copy(z_hbm, acc.at[pl.ds(sid * SP, SP)])

  hc = NCHUNK // 2
  for h in range(2):
    base = wid * NCHUNK + h * hc
    pltpu.sync_copy(edges_hbm.at[0, pl.ds(base, hc)], src_v)
    pltpu.sync_copy(edges_hbm.at[1, pl.ds(base, hc)], dst_v)
    if h == 0:
      plsc.subcore_barrier()
    pltpu.async_copy(g_hbm.at[src_v.at[0]], rows0, gs0)
    pltpu.async_copy(g_hbm.at[src_v.at[1]], rows1, gs1)

    def chunk(j, buf, gs):
      pltpu.make_async_copy(g_hbm.at[src_v.at[j]], buf, gs).wait()
      pltpu.sync_copy(buf, acc.at[dst_v.at[j]], add=True)

      @pl.when(j + 2 < hc)
      def _():
        pltpu.async_copy(g_hbm.at[src_v.at[j + 2]], buf, gs)

    def body(j, carry):
      @pl.when(lax.rem(j, 2) == 0)
      def _():
        chunk(j, rows0, gs0)

      @pl.when(lax.rem(j, 2) == 1)
      def _():
        chunk(j, rows1, gs1)

      return carry

    lax.fori_loop(0, hc, body, 0)

  plsc.subcore_barrier()
  pltpu.sync_copy(acc.at[pl.ds(sid * SP, SP)],
                  out_hbm.at[cid, pl.ds(sid * SP, SP)])


def _prop128_call(g, edges3, z128):
  hc = NCHUNK // 2
  return pl.kernel(
      _prop128_body,
      out_type=jax.ShapeDtypeStruct((NC, NP, D), jnp.float32),
      mesh=_MESH,
      scratch_types=[
          pltpu.VMEM((hc, K), jnp.int32),
          pltpu.VMEM((hc, K), jnp.int32),
          pltpu.VMEM((K, D), jnp.float32),
          pltpu.VMEM((K, D), jnp.float32),
          pltpu.VMEM_SHARED((NP, D), jnp.float32),
          pltpu.SemaphoreType.DMA,
          pltpu.SemaphoreType.DMA,
      ],
      compiler_params=_SC_PARAMS,
  )(g, edges3, z128)


def _prop48_body(g_hbm, edges_hbm, z_hbm, out_hbm,
                 src_v, dst_v, rows0, rows1, rows2, rows3, acc,
                 gs0, gs1, gs2, gs3, ss0, ss1, ss2, ss3):
  cid = lax.axis_index("c")
  sid = lax.axis_index("s")
  wid = sid * NC + cid
  bufs = (rows0, rows1, rows2, rows3)
  gss = (gs0, gs1, gs2, gs3)
  sss = (ss0, ss1, ss2, ss3)
  pltpu.sync_copy(z_hbm, acc.at[pl.ds(sid * SP, SP)])
  base = wid * NCHUNK
  pltpu.sync_copy(edges_hbm.at[0, pl.ds(base, NCHUNK)], src_v)
  pltpu.sync_copy(edges_hbm.at[1, pl.ds(base, NCHUNK)], dst_v)
  plsc.subcore_barrier()

  pltpu.async_copy(g_hbm.at[src_v.at[0]], rows0, gs0)
  pltpu.async_copy(g_hbm.at[src_v.at[1]], rows1, gs1)

  def chunk(j, m):
    pltpu.make_async_copy(g_hbm.at[src_v.at[j]], bufs[m], gss[m]).wait()
    pltpu.async_copy(bufs[m], acc.at[dst_v.at[j]], sss[m], add=True)
    m2 = (m + 2) % 4

    @pl.when(j + 2 < NCHUNK)
    def _():
      @pl.when(j >= 2)
      def _():
        pltpu.make_async_copy(bufs[m2], acc.at[dst_v.at[0]], sss[m2]).wait()

      pltpu.async_copy(g_hbm.at[src_v.at[j + 2]], bufs[m2], gss[m2])

  def body(j, carry):
    m = lax.rem(j, 4)
    for mm in range(4):
      @pl.when(m == mm)
      def _():
        chunk(j, mm)

    return carry

  lax.fori_loop(0, NCHUNK, body, 0)
  pltpu.make_async_copy(bufs[(NCHUNK - 2) % 4], acc.at[dst_v.at[0]],
                        sss[(NCHUNK - 2) % 4]).wait()
  pltpu.make_async_copy(bufs[(NCHUNK - 1) % 4], acc.at[dst_v.at[0]],
                        sss[(NCHUNK - 1) % 4]).wait()
  plsc.subcore_barrier()
  pltpu.sync_copy(acc.at[pl.ds(sid * SP, SP)],
                  out_hbm.at[cid, pl.ds(sid * SP, SP), pl.ds(0, CP)])


def _prop48_call(g, edges3, z48):
  return pl.kernel(
      _prop48_body,
      out_type=jax.ShapeDtypeStruct((NC, NP, D), jnp.float32),
      mesh=_MESH,
      scratch_types=[
          pltpu.VMEM((NCHUNK, K), jnp.int32),
          pltpu.VMEM((NCHUNK, K), jnp.int32),
          pltpu.VMEM((K, CP), jnp.float32),
          pltpu.VMEM((K, CP), jnp.float32),
          pltpu.VMEM((K, CP), jnp.float32),
          pltpu.VMEM((K, CP), jnp.float32),
          pltpu.VMEM_SHARED((NP, CP), jnp.float32),
          pltpu.SemaphoreType.DMA,
          pltpu.SemaphoreType.DMA,
          pltpu.SemaphoreType.DMA,
          pltpu.SemaphoreType.DMA,
          pltpu.SemaphoreType.DMA,
          pltpu.SemaphoreType.DMA,
          pltpu.SemaphoreType.DMA,
          pltpu.SemaphoreType.DMA,
      ],
      compiler_params=_SC_PARAMS,
  )(g, edges3, z48)


def _tc0_kernel(x_ref, w1_ref, b1_ref, h1_ref):
  h = jnp.dot(x_ref[...], w1_ref[...], preferred_element_type=jnp.float32)
  h1_ref[...] = h + b1_ref[...]


def _tc1_kernel(h1_ref, degp_ref, g1_ref, dinvw_ref):
  deg = degp_ref[0, :, 0] + degp_ref[1, :, 0]
  dinvw = lax.rsqrt(jnp.maximum(deg, 1.0))[:, None] * jnp.ones(
      (1, D), jnp.float32)
  dinvw_ref[...] = dinvw
  g1_ref[...] = h1_ref[...] * dinvw


def _tc2_kernel(s1p_ref, dinvw_ref, w2_ref, b2_ref, g2_ref):
  dinvw = dinvw_ref[...]
  h = jnp.maximum((s1p_ref[0] + s1p_ref[1]) * dinvw, 0.0)
  out = jnp.dot(h, w2_ref[...], preferred_element_type=jnp.float32)
  g2_ref[...] = (out + b2_ref[...]) * dinvw[:, :CP]


def _tc3_kernel(s2p_ref, dinvw_ref, out_ref):
  z = (s2p_ref[0, :, :C] + s2p_ref[1, :, :C]) * dinvw_ref[:, :C]
  m = jnp.max(z, axis=1, keepdims=True)
  e = jnp.exp(z - m)
  lse = jnp.log(jnp.sum(e, axis=1, keepdims=True))
  out_ref[...] = z - m - lse


def _tc0(x, W1, b1r):
  return pl.pallas_call(
      _tc0_kernel,
      grid=(N // R,),
      in_specs=[
          pl.BlockSpec((R, D), lambda i: (i, 0)),
          pl.BlockSpec((D, D), lambda i: (0, 0)),
          pl.BlockSpec((1, D), lambda i: (0, 0)),
      ],
      out_specs=pl.BlockSpec((R, D), lambda i: (i, 0)),
      out_shape=jax.ShapeDtypeStruct((N, D), jnp.float32),
  )(x, W1, b1r)


def _tc1(h1, degp):
  return pl.pallas_call(
      _tc1_kernel,
      grid=(N // R,),
      in_specs=[
          pl.BlockSpec((R, D), lambda i: (i, 0)),
          pl.BlockSpec((NC, R, D), lambda i: (0, i, 0)),
      ],
      out_specs=[
          pl.BlockSpec((R, D), lambda i: (i, 0)),
          pl.BlockSpec((R, D), lambda i: (i, 0)),
      ],
      out_shape=[
          jax.ShapeDtypeStruct((N, D), jnp.float32),
          jax.ShapeDtypeStruct((N, D), jnp.float32),
      ],
  )(h1, degp)


def _tc2(s1p, dinvw, W2p, b2p):
  return pl.pallas_call(
      _tc2_kernel,
      grid=(N // R,),
      in_specs=[
          pl.BlockSpec((NC, R, D), lambda i: (0, i, 0)),
          pl.BlockSpec((R, D), lambda i: (i, 0)),
          pl.BlockSpec((D, CP), lambda i: (0, 0)),
          pl.BlockSpec((1, CP), lambda i: (0, 0)),
      ],
      out_specs=pl.BlockSpec((R, CP), lambda i: (i, 0)),
      out_shape=jax.ShapeDtypeStruct((N, CP), jnp.float32),
  )(s1p, dinvw, W2p, b2p)


def _tc3(s2p, dinvw):
  return pl.pallas_call(
      _tc3_kernel,
      grid=(N // R,),
      in_specs=[
          pl.BlockSpec((NC, R, D), lambda i: (0, i, 0)),
          pl.BlockSpec((R, D), lambda i: (i, 0)),
      ],
      out_specs=pl.BlockSpec((R, C), lambda i: (i, 0)),
      out_shape=jax.ShapeDtypeStruct((N, C), jnp.float32),
  )(s2p, dinvw)


_PAD_EDGES = np.stack([(np.arange(EP - E) * 131) % N,
                       N + (np.arange(EP - E) % 16)]).astype(np.int32)


def kernel(x, edge_index, W1, b1, W2, b2):
  pad = jnp.asarray(_PAD_EDGES)
  edges3 = jnp.concatenate([edge_index, pad], axis=1).reshape(2, EP // K, K)
  b1r = b1.reshape(1, D)
  W2p = jnp.pad(W2, ((0, 0), (0, CP - C)))
  b2p = jnp.pad(b2, (0, CP - C)).reshape(1, CP)
  ones8 = jnp.ones((K, 8), jnp.float32)
  z8 = jnp.zeros((SP, 8), jnp.float32)
  z128 = jnp.zeros((SP, D), jnp.float32)
  z48 = jnp.zeros((SP, CP), jnp.float32)

  h1 = _tc0(x, W1, b1r)
  degp = _deg_call(edges3, ones8, z8)
  g1, dinvw = _tc1(h1, degp)
  s1p = _prop128_call(g1, edges3, z128)
  g2 = _tc2(s1p, dinvw, W2p, b2p)
  s2p = _prop48_call(g2, edges3, z48)
  return _tc3(s2p, dinvw)
